# baseline (speedup 1.0000x reference)
import functools

import jax
import jax.numpy as jnp
from jax import lax
from jax.experimental import pallas as pl
from jax.experimental.pallas import tpu as pltpu
from jax.experimental.pallas import tpu_sc as plsc

N = 10000
E = 320000
DIN = 128
H = 64
NG = 256

NC = 2
NS = 16
NW = NC * NS

ECH = 128
ECK = 80
EPW = ECH * ECK
E_PAD = EPW * NW

NROW = N + 112
ZR = NROW // NS

NBUF = 8
LOOK = 4

PCH = 64
PCK = 5
NPW = PCH * PCK
N_PAD = NPW * NW
GROW = 384
GZ = GROW // NS
GW = NG // NS

_mesh = plsc.VectorSubcoreMesh(core_axis_name="c", subcore_axis_name="s")
_sc_params = pltpu.CompilerParams(use_tc_tiling_on_sc=False)


@functools.partial(
    pl.kernel,
    out_type=jax.ShapeDtypeStruct((NC, NROW, 16), jnp.float32),
    mesh=_mesh,
    scratch_types=[
        pltpu.VMEM((ECK, ECH), jnp.int32),
        pltpu.VMEM((ECH, 16), jnp.float32),
        pltpu.VMEM_SHARED((NROW, 16), jnp.float32),
    ],
    compiler_params=_sc_params,
)
def _sc_degree(dstp_hbm, z16_hbm, ones_hbm, out_hbm, didx, ones, acc):
    cid = lax.axis_index("c")
    sid = lax.axis_index("s")
    wid = cid * NS + sid
    pltpu.sync_copy(z16_hbm, acc.at[pl.ds(sid * ZR, ZR)])
    pltpu.sync_copy(ones_hbm, ones)
    pltpu.sync_copy(dstp_hbm.at[pl.ds(wid * ECK, ECK)], didx)
    plsc.subcore_barrier()

    def body(j, carry):
        pltpu.sync_copy(ones, acc.at[didx.at[j]], add=True)
        return carry

    lax.fori_loop(0, ECK, body, 0)
    plsc.subcore_barrier()
    pltpu.sync_copy(acc.at[pl.ds(sid * ZR, ZR)],
                    out_hbm.at[cid, pl.ds(sid * ZR, ZR)])


@functools.partial(
    pl.kernel,
    out_type=jax.ShapeDtypeStruct((NC, NROW, H), jnp.float32),
    mesh=_mesh,
    scratch_types=[
        pltpu.VMEM((ECK, ECH), jnp.int32),
        pltpu.VMEM((ECK, ECH), jnp.int32),
        [pltpu.VMEM((ECH, H), jnp.float32)] * NBUF,
        [pltpu.SemaphoreType.DMA] * NBUF,
        [pltpu.SemaphoreType.DMA] * NBUF,
        pltpu.VMEM_SHARED((NROW, H), jnp.float32),
    ],
    compiler_params=_sc_params,
)
def _sc_propagate(g_hbm, srcp_hbm, dstp_hbm, z64_hbm, out_hbm,
                  sidx, didx, rows, semg, sems, acc):
    cid = lax.axis_index("c")
    sid = lax.axis_index("s")
    wid = cid * NS + sid
    pltpu.sync_copy(z64_hbm, acc.at[pl.ds(sid * ZR, ZR)])
    pltpu.sync_copy(srcp_hbm.at[pl.ds(wid * ECK, ECK)], sidx)
    pltpu.sync_copy(dstp_hbm.at[pl.ds(wid * ECK, ECK)], didx)
    plsc.subcore_barrier()

    for k in range(LOOK):
        pltpu.async_copy(g_hbm.at[sidx.at[k]], rows[k], semg[k])

    def body(i, carry):
        for k in range(NBUF):
            j = i * NBUF + k
            kn = (k + LOOK) % NBUF
            pltpu.make_async_copy(g_hbm.at[sidx.at[j]], rows[k], semg[k]).wait()

            @pl.when(j >= LOOK)
            def _():
                pltpu.make_async_copy(
                    rows[kn], acc.at[didx.at[j]], sems[kn]).wait()

            pltpu.async_copy(rows[k], acc.at[didx.at[j]], sems[k], add=True)

            @pl.when(j + LOOK < ECK)
            def _():
                pltpu.async_copy(g_hbm.at[sidx.at[j + LOOK]], rows[kn], semg[kn])
        return carry

    lax.fori_loop(0, ECK // NBUF, body, 0)
    for k in range(LOOK, NBUF):
        pltpu.make_async_copy(rows[k], acc.at[didx.at[0]], sems[k]).wait()
    plsc.subcore_barrier()
    pltpu.sync_copy(acc.at[pl.ds(sid * ZR, ZR)],
                    out_hbm.at[cid, pl.ds(sid * ZR, ZR)])


@functools.partial(
    pl.kernel,
    out_type=(jax.ShapeDtypeStruct((NC, NG, H), jnp.float32),
              jax.ShapeDtypeStruct((NC, NG, 16), jnp.float32)),
    mesh=_mesh,
    scratch_types=[
        pltpu.VMEM((8, PCH), jnp.int32),
        pltpu.VMEM((PCH, H), jnp.float32),
        pltpu.VMEM((PCH, 16), jnp.float32),
        pltpu.VMEM_SHARED((GROW, H), jnp.float32),
        pltpu.VMEM_SHARED((GROW, 16), jnp.float32),
    ],
    compiler_params=_sc_params,
)
def _sc_pool(h_hbm, bidp_hbm, z64_hbm, z16_hbm, ones_hbm, outp_hbm, outc_hbm,
             bidx, rows, ones, accp, accc):
    cid = lax.axis_index("c")
    sid = lax.axis_index("s")
    wid = cid * NS + sid
    pltpu.sync_copy(z64_hbm.at[pl.ds(0, GZ)], accp.at[pl.ds(sid * GZ, GZ)])
    pltpu.sync_copy(z16_hbm.at[pl.ds(0, GZ)], accc.at[pl.ds(sid * GZ, GZ)])
    pltpu.sync_copy(ones_hbm.at[pl.ds(0, PCH)], ones)
    pltpu.sync_copy(bidp_hbm.at[pl.ds(wid * 8, 8)], bidx)
    plsc.subcore_barrier()

    def body(j, carry):
        pltpu.sync_copy(h_hbm.at[pl.ds(wid * NPW + j * PCH, PCH)], rows)
        pltpu.sync_copy(rows, accp.at[bidx.at[j]], add=True)
        pltpu.sync_copy(ones, accc.at[bidx.at[j]], add=True)
        return carry

    lax.fori_loop(0, PCK, body, 0)
    plsc.subcore_barrier()
    pltpu.sync_copy(accp.at[pl.ds(sid * GW, GW)],
                    outp_hbm.at[cid, pl.ds(sid * GW, GW)])
    pltpu.sync_copy(accc.at[pl.ds(sid * GW, GW)],
                    outc_hbm.at[cid, pl.ds(sid * GW, GW)])


def _tc_g0_body(x_ref, w_ref, deg_ref, g_ref, dinv_ref):
    deg = deg_ref[0, :N, 0:1] + deg_ref[1, :N, 0:1] + 1.0
    dinv = lax.rsqrt(deg)
    g_ref[...] = dinv * jnp.dot(x_ref[...], w_ref[...],
                                preferred_element_type=jnp.float32)
    dinv_ref[...] = dinv


_tc_g0 = pl.pallas_call(
    _tc_g0_body,
    out_shape=(jax.ShapeDtypeStruct((N, H), jnp.float32),
               jax.ShapeDtypeStruct((N, 1), jnp.float32)),
)


def _tc_epi_body(s_ref, g_ref, dinv_ref, b_ref, w_ref, out_ref):
    dinv = dinv_ref[...]
    h = jnp.maximum(
        dinv * (s_ref[0, :N] + s_ref[1, :N] + g_ref[...]) + b_ref[...], 0.0)
    out_ref[...] = dinv * jnp.dot(h, w_ref[...],
                                  preferred_element_type=jnp.float32)


_tc_epi = pl.pallas_call(
    _tc_epi_body,
    out_shape=jax.ShapeDtypeStruct((N, H), jnp.float32),
)


def _tc_last_body(s_ref, g_ref, dinv_ref, b_ref, out_ref):
    dinv = dinv_ref[...]
    out_ref[...] = jnp.maximum(
        dinv * (s_ref[0, :N] + s_ref[1, :N] + g_ref[...]) + b_ref[...], 0.0)


_tc_last = pl.pallas_call(
    _tc_last_body,
    out_shape=jax.ShapeDtypeStruct((N, H), jnp.float32),
)


def _tc_head_body(p_ref, c_ref, wo_ref, bo_ref, out_ref, hid_ref):
    cnt = jnp.maximum(c_ref[0, :, 0:1] + c_ref[1, :, 0:1], 1.0)
    pooled = (p_ref[0] + p_ref[1]) / cnt
    hid_ref[...] = pooled
    out_ref[...] = jnp.dot(pooled, wo_ref[...],
                           preferred_element_type=jnp.float32) + bo_ref[...]


_tc_head = pl.pallas_call(
    _tc_head_body,
    out_shape=(jax.ShapeDtypeStruct((NG, 1), jnp.float32),
               jax.ShapeDtypeStruct((NG, H), jnp.float32)),
)


def kernel(x, edge_index, batch_index, W0, b0, W1, b1, W2, b2, W3, b3,
           W_out, b_out):
    src = edge_index[0]
    dst = edge_index[1]
    pad_dst = N + (jnp.arange(E_PAD - E, dtype=jnp.int32) % (NROW - N))
    srcp = jnp.concatenate(
        [src, jnp.zeros((E_PAD - E,), jnp.int32)]).reshape(NW * ECK, ECH)
    dstp = jnp.concatenate([dst, pad_dst]).reshape(NW * ECK, ECH)
    pad_bid = NG + (jnp.arange(N_PAD - N, dtype=jnp.int32) % (GROW - NG - 8))
    bidw = jnp.concatenate(
        [batch_index, pad_bid]).reshape(NW, PCK, PCH)
    bidp = jnp.concatenate(
        [bidw, jnp.full((NW, 8 - PCK, PCH), NG, jnp.int32)],
        axis=1).reshape(NW * 8, PCH)
    z64 = jnp.zeros((ZR, H), jnp.float32)
    z16 = jnp.zeros((ZR, 16), jnp.float32)
    ones16 = jnp.ones((ECH, 16), jnp.float32)

    degp = _sc_degree(dstp, z16, ones16)
    g, dinv = _tc_g0(x, W0, degp)

    bs = (b0, b1, b2, b3)
    nxt = (W1, W2, W3)
    for l in range(4):
        S = _sc_propagate(g, srcp, dstp, z64)
        if l < 3:
            g = _tc_epi(S, g, dinv, bs[l].reshape(1, H), nxt[l])
        else:
            h3 = _tc_last(S, g, dinv, bs[3].reshape(1, H))

    h3p = jnp.pad(h3, ((0, N_PAD - N), (0, 0)))
    P, C = _sc_pool(h3p, bidp, z64, z16, ones16)
    out, hid = _tc_head(P, C, W_out, b_out.reshape(1, 1))
    return (out, hid)


# speedup vs baseline: 1.0004x; 1.0004x over previous
import functools

import jax
import jax.numpy as jnp
from jax import lax
from jax.experimental import pallas as pl
from jax.experimental.pallas import tpu as pltpu
from jax.experimental.pallas import tpu_sc as plsc

N = 10000
E = 320000
DIN = 128
H = 64
NG = 256

NC = 2
NS = 16
NW = NC * NS

ECH = 128
ECK = 80
EPW = ECH * ECK
E_PAD = EPW * NW

NROW = N + 112
ZR = NROW // NS

NBUF = 8
LOOK = 4

PCH = 64
PCK = 5
NPW = PCH * PCK
N_PAD = NPW * NW
GROW = 384
GZ = GROW // NS
GW = NG // NS

_mesh = plsc.VectorSubcoreMesh(core_axis_name="c", subcore_axis_name="s")
_sc_params = pltpu.CompilerParams(use_tc_tiling_on_sc=False)


@functools.partial(
    pl.kernel,
    out_type=jax.ShapeDtypeStruct((NC, NROW, 16), jnp.float32),
    mesh=_mesh,
    scratch_types=[
        pltpu.VMEM((ECK, ECH), jnp.int32),
        pltpu.VMEM((ECH, 16), jnp.float32),
        pltpu.VMEM_SHARED((NROW, 16), jnp.float32),
    ],
    compiler_params=_sc_params,
)
def _sc_degree(dstp_hbm, z16_hbm, ones_hbm, out_hbm, didx, ones, acc):
    cid = lax.axis_index("c")
    sid = lax.axis_index("s")
    wid = cid * NS + sid
    pltpu.sync_copy(z16_hbm, acc.at[pl.ds(sid * ZR, ZR)])
    pltpu.sync_copy(ones_hbm, ones)
    pltpu.sync_copy(dstp_hbm.at[pl.ds(wid * ECK, ECK)], didx)
    plsc.subcore_barrier()

    def body(j, carry):
        pltpu.sync_copy(ones, acc.at[didx.at[j]], add=True)
        return carry

    lax.fori_loop(0, ECK, body, 0)
    plsc.subcore_barrier()
    pltpu.sync_copy(acc.at[pl.ds(sid * ZR, ZR)],
                    out_hbm.at[cid, pl.ds(sid * ZR, ZR)])


@functools.partial(
    pl.kernel,
    out_type=jax.ShapeDtypeStruct((NC, NROW, H), jnp.float32),
    mesh=_mesh,
    scratch_types=[
        pltpu.VMEM((ECK, ECH), jnp.int32),
        pltpu.VMEM((ECK, ECH), jnp.int32),
        [pltpu.VMEM((ECH, H), jnp.float32)] * NBUF,
        [pltpu.SemaphoreType.DMA] * NBUF,
        [pltpu.SemaphoreType.DMA] * NBUF,
        pltpu.VMEM_SHARED((NROW, H), jnp.float32),
    ],
    compiler_params=_sc_params,
)
def _sc_propagate(g_hbm, srcp_hbm, dstp_hbm, z64_hbm, out_hbm,
                  sidx, didx, rows, semg, sems, acc):
    cid = lax.axis_index("c")
    sid = lax.axis_index("s")
    wid = cid * NS + sid
    pltpu.sync_copy(z64_hbm, acc.at[pl.ds(sid * ZR, ZR)])
    pltpu.sync_copy(srcp_hbm.at[pl.ds(wid * ECK, ECK)], sidx)
    pltpu.sync_copy(dstp_hbm.at[pl.ds(wid * ECK, ECK)], didx)
    plsc.subcore_barrier()

    for k in range(LOOK):
        pltpu.async_copy(g_hbm.at[sidx.at[k]], rows[k], semg[k])

    def body(i, carry):
        for k in range(NBUF):
            j = i * NBUF + k
            kn = (k + LOOK) % NBUF
            kp = (k + NBUF - 2) % NBUF
            pltpu.make_async_copy(g_hbm.at[sidx.at[j]], rows[k], semg[k]).wait()

            @pl.when(j >= 2)
            def _():
                pltpu.make_async_copy(
                    rows[kp], acc.at[didx.at[j]], sems[kp]).wait()

            pltpu.async_copy(rows[k], acc.at[didx.at[j]], sems[k], add=True)

            @pl.when(j + LOOK < ECK)
            def _():
                pltpu.async_copy(g_hbm.at[sidx.at[j + LOOK]], rows[kn], semg[kn])
        return carry

    lax.fori_loop(0, ECK // NBUF, body, 0)
    for k in range(NBUF - 2, NBUF):
        pltpu.make_async_copy(rows[k], acc.at[didx.at[0]], sems[k]).wait()
    plsc.subcore_barrier()
    pltpu.sync_copy(acc.at[pl.ds(sid * ZR, ZR)],
                    out_hbm.at[cid, pl.ds(sid * ZR, ZR)])


@functools.partial(
    pl.kernel,
    out_type=(jax.ShapeDtypeStruct((NC, NG, H), jnp.float32),
              jax.ShapeDtypeStruct((NC, NG, 16), jnp.float32)),
    mesh=_mesh,
    scratch_types=[
        pltpu.VMEM((8, PCH), jnp.int32),
        pltpu.VMEM((PCH, H), jnp.float32),
        pltpu.VMEM((PCH, 16), jnp.float32),
        pltpu.VMEM_SHARED((GROW, H), jnp.float32),
        pltpu.VMEM_SHARED((GROW, 16), jnp.float32),
    ],
    compiler_params=_sc_params,
)
def _sc_pool(h_hbm, bidp_hbm, z64_hbm, z16_hbm, ones_hbm, outp_hbm, outc_hbm,
             bidx, rows, ones, accp, accc):
    cid = lax.axis_index("c")
    sid = lax.axis_index("s")
    wid = cid * NS + sid
    pltpu.sync_copy(z64_hbm.at[pl.ds(0, GZ)], accp.at[pl.ds(sid * GZ, GZ)])
    pltpu.sync_copy(z16_hbm.at[pl.ds(0, GZ)], accc.at[pl.ds(sid * GZ, GZ)])
    pltpu.sync_copy(ones_hbm.at[pl.ds(0, PCH)], ones)
    pltpu.sync_copy(bidp_hbm.at[pl.ds(wid * 8, 8)], bidx)
    plsc.subcore_barrier()

    def body(j, carry):
        pltpu.sync_copy(h_hbm.at[pl.ds(wid * NPW + j * PCH, PCH)], rows)
        pltpu.sync_copy(rows, accp.at[bidx.at[j]], add=True)
        pltpu.sync_copy(ones, accc.at[bidx.at[j]], add=True)
        return carry

    lax.fori_loop(0, PCK, body, 0)
    plsc.subcore_barrier()
    pltpu.sync_copy(accp.at[pl.ds(sid * GW, GW)],
                    outp_hbm.at[cid, pl.ds(sid * GW, GW)])
    pltpu.sync_copy(accc.at[pl.ds(sid * GW, GW)],
                    outc_hbm.at[cid, pl.ds(sid * GW, GW)])


def _tc_g0_body(x_ref, w_ref, deg_ref, g_ref, dinv_ref):
    deg = deg_ref[0, :N, 0:1] + deg_ref[1, :N, 0:1] + 1.0
    dinv = lax.rsqrt(deg)
    g_ref[...] = dinv * jnp.dot(x_ref[...], w_ref[...],
                                preferred_element_type=jnp.float32)
    dinv_ref[...] = dinv


_tc_g0 = pl.pallas_call(
    _tc_g0_body,
    out_shape=(jax.ShapeDtypeStruct((N, H), jnp.float32),
               jax.ShapeDtypeStruct((N, 1), jnp.float32)),
)


def _tc_epi_body(s_ref, g_ref, dinv_ref, b_ref, w_ref, out_ref):
    dinv = dinv_ref[...]
    h = jnp.maximum(
        dinv * (s_ref[0, :N] + s_ref[1, :N] + g_ref[...]) + b_ref[...], 0.0)
    out_ref[...] = dinv * jnp.dot(h, w_ref[...],
                                  preferred_element_type=jnp.float32)


_tc_epi = pl.pallas_call(
    _tc_epi_body,
    out_shape=jax.ShapeDtypeStruct((N, H), jnp.float32),
)


def _tc_last_body(s_ref, g_ref, dinv_ref, b_ref, out_ref):
    dinv = dinv_ref[...]
    out_ref[...] = jnp.maximum(
        dinv * (s_ref[0, :N] + s_ref[1, :N] + g_ref[...]) + b_ref[...], 0.0)


_tc_last = pl.pallas_call(
    _tc_last_body,
    out_shape=jax.ShapeDtypeStruct((N, H), jnp.float32),
)


def _tc_head_body(p_ref, c_ref, wo_ref, bo_ref, out_ref, hid_ref):
    cnt = jnp.maximum(c_ref[0, :, 0:1] + c_ref[1, :, 0:1], 1.0)
    pooled = (p_ref[0] + p_ref[1]) / cnt
    hid_ref[...] = pooled
    out_ref[...] = jnp.dot(pooled, wo_ref[...],
                           preferred_element_type=jnp.float32) + bo_ref[...]


_tc_head = pl.pallas_call(
    _tc_head_body,
    out_shape=(jax.ShapeDtypeStruct((NG, 1), jnp.float32),
               jax.ShapeDtypeStruct((NG, H), jnp.float32)),
)


def kernel(x, edge_index, batch_index, W0, b0, W1, b1, W2, b2, W3, b3,
           W_out, b_out):
    src = edge_index[0]
    dst = edge_index[1]
    pad_dst = N + (jnp.arange(E_PAD - E, dtype=jnp.int32) % (NROW - N))
    srcp = jnp.concatenate(
        [src, jnp.zeros((E_PAD - E,), jnp.int32)]).reshape(NW * ECK, ECH)
    dstp = jnp.concatenate([dst, pad_dst]).reshape(NW * ECK, ECH)
    pad_bid = NG + (jnp.arange(N_PAD - N, dtype=jnp.int32) % (GROW - NG - 8))
    bidw = jnp.concatenate(
        [batch_index, pad_bid]).reshape(NW, PCK, PCH)
    bidp = jnp.concatenate(
        [bidw, jnp.full((NW, 8 - PCK, PCH), NG, jnp.int32)],
        axis=1).reshape(NW * 8, PCH)
    z64 = jnp.zeros((ZR, H), jnp.float32)
    z16 = jnp.zeros((ZR, 16), jnp.float32)
    ones16 = jnp.ones((ECH, 16), jnp.float32)

    degp = _sc_degree(dstp, z16, ones16)
    g, dinv = _tc_g0(x, W0, degp)

    bs = (b0, b1, b2, b3)
    nxt = (W1, W2, W3)
    for l in range(4):
        S = _sc_propagate(g, srcp, dstp, z64)
        if l < 3:
            g = _tc_epi(S, g, dinv, bs[l].reshape(1, H), nxt[l])
        else:
            h3 = _tc_last(S, g, dinv, bs[3].reshape(1, H))

    h3p = jnp.pad(h3, ((0, N_PAD - N), (0, 0)))
    P, C = _sc_pool(h3p, bidp, z64, z16, ones16)
    out, hid = _tc_head(P, C, W_out, b_out.reshape(1, 1))
    return (out, hid)


# speedup vs baseline: 1.0006x; 1.0001x over previous
import functools

import jax
import jax.numpy as jnp
from jax import lax
from jax.experimental import pallas as pl
from jax.experimental.pallas import tpu as pltpu
from jax.experimental.pallas import tpu_sc as plsc

N = 10000
E = 320000
DIN = 128
H = 64
NG = 256

NC = 2
NS = 16
NW = NC * NS

ECH = 256
ECK = 40
EPW = ECH * ECK
E_PAD = EPW * NW

NROW = N + 112
ZR = NROW // NS

NBUF = 4
LOOK = 2

PCH = 64
PCK = 5
NPW = PCH * PCK
N_PAD = NPW * NW
GROW = 384
GZ = GROW // NS
GW = NG // NS

_mesh = plsc.VectorSubcoreMesh(core_axis_name="c", subcore_axis_name="s")
_sc_params = pltpu.CompilerParams(use_tc_tiling_on_sc=False)


@functools.partial(
    pl.kernel,
    out_type=jax.ShapeDtypeStruct((NC, NROW, 16), jnp.float32),
    mesh=_mesh,
    scratch_types=[
        pltpu.VMEM((ECK, ECH), jnp.int32),
        pltpu.VMEM((ECH, 16), jnp.float32),
        pltpu.VMEM_SHARED((NROW, 16), jnp.float32),
    ],
    compiler_params=_sc_params,
)
def _sc_degree(dstp_hbm, z16_hbm, ones_hbm, out_hbm, didx, ones, acc):
    cid = lax.axis_index("c")
    sid = lax.axis_index("s")
    wid = cid * NS + sid
    pltpu.sync_copy(z16_hbm, acc.at[pl.ds(sid * ZR, ZR)])
    pltpu.sync_copy(ones_hbm, ones)
    pltpu.sync_copy(dstp_hbm.at[pl.ds(wid * ECK, ECK)], didx)
    plsc.subcore_barrier()

    def body(j, carry):
        pltpu.sync_copy(ones, acc.at[didx.at[j]], add=True)
        return carry

    lax.fori_loop(0, ECK, body, 0)
    plsc.subcore_barrier()
    pltpu.sync_copy(acc.at[pl.ds(sid * ZR, ZR)],
                    out_hbm.at[cid, pl.ds(sid * ZR, ZR)])


@functools.partial(
    pl.kernel,
    out_type=jax.ShapeDtypeStruct((NC, NROW, H), jnp.float32),
    mesh=_mesh,
    scratch_types=[
        pltpu.VMEM((ECK, ECH), jnp.int32),
        pltpu.VMEM((ECK, ECH), jnp.int32),
        [pltpu.VMEM((ECH, H), jnp.float32)] * NBUF,
        [pltpu.SemaphoreType.DMA] * NBUF,
        [pltpu.SemaphoreType.DMA] * NBUF,
        pltpu.VMEM_SHARED((NROW, H), jnp.float32),
    ],
    compiler_params=_sc_params,
)
def _sc_propagate(g_hbm, srcp_hbm, dstp_hbm, z64_hbm, out_hbm,
                  sidx, didx, rows, semg, sems, acc):
    cid = lax.axis_index("c")
    sid = lax.axis_index("s")
    wid = cid * NS + sid
    pltpu.sync_copy(z64_hbm, acc.at[pl.ds(sid * ZR, ZR)])
    pltpu.sync_copy(srcp_hbm.at[pl.ds(wid * ECK, ECK)], sidx)
    pltpu.sync_copy(dstp_hbm.at[pl.ds(wid * ECK, ECK)], didx)
    plsc.subcore_barrier()

    for k in range(LOOK):
        pltpu.async_copy(g_hbm.at[sidx.at[k]], rows[k], semg[k])

    def body(i, carry):
        for k in range(NBUF):
            j = i * NBUF + k
            kn = (k + LOOK) % NBUF
            pltpu.make_async_copy(g_hbm.at[sidx.at[j]], rows[k], semg[k]).wait()

            @pl.when(j >= LOOK)
            def _():
                pltpu.make_async_copy(
                    rows[kn], acc.at[didx.at[j]], sems[kn]).wait()

            pltpu.async_copy(rows[k], acc.at[didx.at[j]], sems[k], add=True)

            @pl.when(j + LOOK < ECK)
            def _():
                pltpu.async_copy(g_hbm.at[sidx.at[j + LOOK]], rows[kn], semg[kn])
        return carry

    lax.fori_loop(0, ECK // NBUF, body, 0)
    for k in range(NBUF - LOOK, NBUF):
        pltpu.make_async_copy(rows[k], acc.at[didx.at[0]], sems[k]).wait()
    plsc.subcore_barrier()
    pltpu.sync_copy(acc.at[pl.ds(sid * ZR, ZR)],
                    out_hbm.at[cid, pl.ds(sid * ZR, ZR)])


@functools.partial(
    pl.kernel,
    out_type=(jax.ShapeDtypeStruct((NC, NG, H), jnp.float32),
              jax.ShapeDtypeStruct((NC, NG, 16), jnp.float32)),
    mesh=_mesh,
    scratch_types=[
        pltpu.VMEM((8, PCH), jnp.int32),
        pltpu.VMEM((PCH, H), jnp.float32),
        pltpu.VMEM((PCH, 16), jnp.float32),
        pltpu.VMEM_SHARED((GROW, H), jnp.float32),
        pltpu.VMEM_SHARED((GROW, 16), jnp.float32),
    ],
    compiler_params=_sc_params,
)
def _sc_pool(h_hbm, bidp_hbm, z64_hbm, z16_hbm, ones_hbm, outp_hbm, outc_hbm,
             bidx, rows, ones, accp, accc):
    cid = lax.axis_index("c")
    sid = lax.axis_index("s")
    wid = cid * NS + sid
    pltpu.sync_copy(z64_hbm.at[pl.ds(0, GZ)], accp.at[pl.ds(sid * GZ, GZ)])
    pltpu.sync_copy(z16_hbm.at[pl.ds(0, GZ)], accc.at[pl.ds(sid * GZ, GZ)])
    pltpu.sync_copy(ones_hbm.at[pl.ds(0, PCH)], ones)
    pltpu.sync_copy(bidp_hbm.at[pl.ds(wid * 8, 8)], bidx)
    plsc.subcore_barrier()

    def body(j, carry):
        pltpu.sync_copy(h_hbm.at[pl.ds(wid * NPW + j * PCH, PCH)], rows)
        pltpu.sync_copy(rows, accp.at[bidx.at[j]], add=True)
        pltpu.sync_copy(ones, accc.at[bidx.at[j]], add=True)
        return carry

    lax.fori_loop(0, PCK, body, 0)
    plsc.subcore_barrier()
    pltpu.sync_copy(accp.at[pl.ds(sid * GW, GW)],
                    outp_hbm.at[cid, pl.ds(sid * GW, GW)])
    pltpu.sync_copy(accc.at[pl.ds(sid * GW, GW)],
                    outc_hbm.at[cid, pl.ds(sid * GW, GW)])


def _tc_g0_body(x_ref, w_ref, deg_ref, g_ref, dinv_ref):
    deg = deg_ref[0, :N, 0:1] + deg_ref[1, :N, 0:1] + 1.0
    dinv = lax.rsqrt(deg)
    g_ref[...] = dinv * jnp.dot(x_ref[...], w_ref[...],
                                preferred_element_type=jnp.float32)
    dinv_ref[...] = dinv


_tc_g0 = pl.pallas_call(
    _tc_g0_body,
    out_shape=(jax.ShapeDtypeStruct((N, H), jnp.float32),
               jax.ShapeDtypeStruct((N, 1), jnp.float32)),
)


def _tc_epi_body(s_ref, g_ref, dinv_ref, b_ref, w_ref, out_ref):
    dinv = dinv_ref[...]
    h = jnp.maximum(
        dinv * (s_ref[0, :N] + s_ref[1, :N] + g_ref[...]) + b_ref[...], 0.0)
    out_ref[...] = dinv * jnp.dot(h, w_ref[...],
                                  preferred_element_type=jnp.float32)


_tc_epi = pl.pallas_call(
    _tc_epi_body,
    out_shape=jax.ShapeDtypeStruct((N, H), jnp.float32),
)


def _tc_last_body(s_ref, g_ref, dinv_ref, b_ref, out_ref):
    dinv = dinv_ref[...]
    out_ref[...] = jnp.maximum(
        dinv * (s_ref[0, :N] + s_ref[1, :N] + g_ref[...]) + b_ref[...], 0.0)


_tc_last = pl.pallas_call(
    _tc_last_body,
    out_shape=jax.ShapeDtypeStruct((N, H), jnp.float32),
)


def _tc_head_body(p_ref, c_ref, wo_ref, bo_ref, out_ref, hid_ref):
    cnt = jnp.maximum(c_ref[0, :, 0:1] + c_ref[1, :, 0:1], 1.0)
    pooled = (p_ref[0] + p_ref[1]) / cnt
    hid_ref[...] = pooled
    out_ref[...] = jnp.dot(pooled, wo_ref[...],
                           preferred_element_type=jnp.float32) + bo_ref[...]


_tc_head = pl.pallas_call(
    _tc_head_body,
    out_shape=(jax.ShapeDtypeStruct((NG, 1), jnp.float32),
               jax.ShapeDtypeStruct((NG, H), jnp.float32)),
)


def kernel(x, edge_index, batch_index, W0, b0, W1, b1, W2, b2, W3, b3,
           W_out, b_out):
    src = edge_index[0]
    dst = edge_index[1]
    pad_dst = N + (jnp.arange(E_PAD - E, dtype=jnp.int32) % (NROW - N))
    srcp = jnp.concatenate(
        [src, jnp.zeros((E_PAD - E,), jnp.int32)]).reshape(NW * ECK, ECH)
    dstp = jnp.concatenate([dst, pad_dst]).reshape(NW * ECK, ECH)
    pad_bid = NG + (jnp.arange(N_PAD - N, dtype=jnp.int32) % (GROW - NG - 8))
    bidw = jnp.concatenate(
        [batch_index, pad_bid]).reshape(NW, PCK, PCH)
    bidp = jnp.concatenate(
        [bidw, jnp.full((NW, 8 - PCK, PCH), NG, jnp.int32)],
        axis=1).reshape(NW * 8, PCH)
    z64 = jnp.zeros((ZR, H), jnp.float32)
    z16 = jnp.zeros((ZR, 16), jnp.float32)
    ones16 = jnp.ones((ECH, 16), jnp.float32)

    degp = _sc_degree(dstp, z16, ones16)
    g, dinv = _tc_g0(x, W0, degp)

    bs = (b0, b1, b2, b3)
    nxt = (W1, W2, W3)
    for l in range(4):
        S = _sc_propagate(g, srcp, dstp, z64)
        if l < 3:
            g = _tc_epi(S, g, dinv, bs[l].reshape(1, H), nxt[l])
        else:
            h3 = _tc_last(S, g, dinv, bs[3].reshape(1, H))

    h3p = jnp.pad(h3, ((0, N_PAD - N), (0, 0)))
    P, C = _sc_pool(h3p, bidp, z64, z16, ones16)
    out, hid = _tc_head(P, C, W_out, b_out.reshape(1, 1))
    return (out, hid)


# speedup vs baseline: 1.0068x; 1.0062x over previous
import functools

import jax
import jax.numpy as jnp
from jax import lax
from jax.experimental import pallas as pl
from jax.experimental.pallas import tpu as pltpu
from jax.experimental.pallas import tpu_sc as plsc

N = 10000
E = 320000
DIN = 128
H = 64
NG = 256

NC = 2
NS = 16
NW = NC * NS

ECH = 128
ECK = 80
EPW = ECH * ECK
E_PAD = EPW * NW

NROW = N + 112
ZR = NROW // NS

NBUF = 4
LOOK = 2

PCH = 64
PCK = 5
NPW = PCH * PCK
N_PAD = NPW * NW
GROW = 384
GZ = GROW // NS
GW = NG // NS

_mesh = plsc.VectorSubcoreMesh(core_axis_name="c", subcore_axis_name="s")
_sc_params = pltpu.CompilerParams(use_tc_tiling_on_sc=False)


@functools.partial(
    pl.kernel,
    out_type=jax.ShapeDtypeStruct((NC, NROW, 16), jnp.float32),
    mesh=_mesh,
    scratch_types=[
        pltpu.VMEM((ECK, ECH), jnp.int32),
        pltpu.VMEM((ECH, 16), jnp.float32),
        pltpu.VMEM_SHARED((NROW, 16), jnp.float32),
    ],
    compiler_params=_sc_params,
)
def _sc_degree(dstp_hbm, z16_hbm, ones_hbm, out_hbm, didx, ones, acc):
    cid = lax.axis_index("c")
    sid = lax.axis_index("s")
    wid = cid * NS + sid
    pltpu.sync_copy(z16_hbm, acc.at[pl.ds(sid * ZR, ZR)])
    pltpu.sync_copy(ones_hbm, ones)
    pltpu.sync_copy(dstp_hbm.at[pl.ds(wid * ECK, ECK)], didx)
    plsc.subcore_barrier()

    def body(j, carry):
        pltpu.sync_copy(ones, acc.at[didx.at[j]], add=True)
        return carry

    lax.fori_loop(0, ECK, body, 0)
    plsc.subcore_barrier()
    pltpu.sync_copy(acc.at[pl.ds(sid * ZR, ZR)],
                    out_hbm.at[cid, pl.ds(sid * ZR, ZR)])


@functools.partial(
    pl.kernel,
    out_type=jax.ShapeDtypeStruct((NC, NROW, H), jnp.float32),
    mesh=_mesh,
    scratch_types=[
        pltpu.VMEM((ECK, ECH), jnp.int32),
        pltpu.VMEM((ECK, ECH), jnp.int32),
        [pltpu.VMEM((ECH, H), jnp.float32)] * NBUF,
        [pltpu.SemaphoreType.DMA] * NBUF,
        [pltpu.SemaphoreType.DMA] * NBUF,
        pltpu.VMEM_SHARED((NROW, H), jnp.float32),
    ],
    compiler_params=_sc_params,
)
def _sc_propagate(g_hbm, srcp_hbm, dstp_hbm, z64_hbm, out_hbm,
                  sidx, didx, rows, semg, sems, acc):
    cid = lax.axis_index("c")
    sid = lax.axis_index("s")
    wid = cid * NS + sid
    pltpu.sync_copy(z64_hbm, acc.at[pl.ds(sid * ZR, ZR)])
    pltpu.sync_copy(srcp_hbm.at[pl.ds(wid * ECK, ECK)], sidx)
    pltpu.sync_copy(dstp_hbm.at[pl.ds(wid * ECK, ECK)], didx)
    plsc.subcore_barrier()

    for k in range(LOOK):
        pltpu.async_copy(g_hbm.at[sidx.at[k]], rows[k], semg[k])

    def body(i, carry):
        for k in range(NBUF):
            j = i * NBUF + k
            kn = (k + LOOK) % NBUF
            pltpu.make_async_copy(g_hbm.at[sidx.at[j]], rows[k], semg[k]).wait()

            @pl.when(j >= LOOK)
            def _():
                pltpu.make_async_copy(
                    rows[kn], acc.at[didx.at[j]], sems[kn]).wait()

            pltpu.async_copy(rows[k], acc.at[didx.at[j]], sems[k], add=True)

            @pl.when(j + LOOK < ECK)
            def _():
                pltpu.async_copy(g_hbm.at[sidx.at[j + LOOK]], rows[kn], semg[kn])
        return carry

    lax.fori_loop(0, ECK // NBUF, body, 0)
    for k in range(NBUF - LOOK, NBUF):
        pltpu.make_async_copy(rows[k], acc.at[didx.at[0]], sems[k]).wait()
    plsc.subcore_barrier()
    pltpu.sync_copy(acc.at[pl.ds(sid * ZR, ZR)],
                    out_hbm.at[cid, pl.ds(sid * ZR, ZR)])


@functools.partial(
    pl.kernel,
    out_type=(jax.ShapeDtypeStruct((NC, NG, H), jnp.float32),
              jax.ShapeDtypeStruct((NC, NG, 16), jnp.float32)),
    mesh=_mesh,
    scratch_types=[
        pltpu.VMEM((8, PCH), jnp.int32),
        pltpu.VMEM((PCH, H), jnp.float32),
        pltpu.VMEM((PCH, 16), jnp.float32),
        pltpu.VMEM_SHARED((GROW, H), jnp.float32),
        pltpu.VMEM_SHARED((GROW, 16), jnp.float32),
    ],
    compiler_params=_sc_params,
)
def _sc_pool(h_hbm, bidp_hbm, z64_hbm, z16_hbm, ones_hbm, outp_hbm, outc_hbm,
             bidx, rows, ones, accp, accc):
    cid = lax.axis_index("c")
    sid = lax.axis_index("s")
    wid = cid * NS + sid
    pltpu.sync_copy(z64_hbm.at[pl.ds(0, GZ)], accp.at[pl.ds(sid * GZ, GZ)])
    pltpu.sync_copy(z16_hbm.at[pl.ds(0, GZ)], accc.at[pl.ds(sid * GZ, GZ)])
    pltpu.sync_copy(ones_hbm.at[pl.ds(0, PCH)], ones)
    pltpu.sync_copy(bidp_hbm.at[pl.ds(wid * 8, 8)], bidx)
    plsc.subcore_barrier()

    def body(j, carry):
        pltpu.sync_copy(h_hbm.at[pl.ds(wid * NPW + j * PCH, PCH)], rows)
        pltpu.sync_copy(rows, accp.at[bidx.at[j]], add=True)
        pltpu.sync_copy(ones, accc.at[bidx.at[j]], add=True)
        return carry

    lax.fori_loop(0, PCK, body, 0)
    plsc.subcore_barrier()
    pltpu.sync_copy(accp.at[pl.ds(sid * GW, GW)],
                    outp_hbm.at[cid, pl.ds(sid * GW, GW)])
    pltpu.sync_copy(accc.at[pl.ds(sid * GW, GW)],
                    outc_hbm.at[cid, pl.ds(sid * GW, GW)])


def _tc_g0_body(x_ref, w_ref, deg_ref, g_ref, dinv_ref):
    deg = deg_ref[0, :N, 0:1] + deg_ref[1, :N, 0:1] + 1.0
    dinv = lax.rsqrt(deg)
    g_ref[...] = dinv * jnp.dot(x_ref[...], w_ref[...],
                                preferred_element_type=jnp.float32)
    dinv_ref[...] = dinv


_tc_g0 = pl.pallas_call(
    _tc_g0_body,
    out_shape=(jax.ShapeDtypeStruct((N, H), jnp.float32),
               jax.ShapeDtypeStruct((N, 1), jnp.float32)),
)


def _tc_epi_body(s_ref, g_ref, dinv_ref, b_ref, w_ref, out_ref):
    dinv = dinv_ref[...]
    h = jnp.maximum(
        dinv * (s_ref[0, :N] + s_ref[1, :N] + g_ref[...]) + b_ref[...], 0.0)
    out_ref[...] = dinv * jnp.dot(h, w_ref[...],
                                  preferred_element_type=jnp.float32)


_tc_epi = pl.pallas_call(
    _tc_epi_body,
    out_shape=jax.ShapeDtypeStruct((N, H), jnp.float32),
)


def _tc_last_body(s_ref, g_ref, dinv_ref, b_ref, out_ref):
    dinv = dinv_ref[...]
    out_ref[...] = jnp.maximum(
        dinv * (s_ref[0, :N] + s_ref[1, :N] + g_ref[...]) + b_ref[...], 0.0)


_tc_last = pl.pallas_call(
    _tc_last_body,
    out_shape=jax.ShapeDtypeStruct((N, H), jnp.float32),
)


def _tc_head_body(p_ref, c_ref, wo_ref, bo_ref, out_ref, hid_ref):
    cnt = jnp.maximum(c_ref[0, :, 0:1] + c_ref[1, :, 0:1], 1.0)
    pooled = (p_ref[0] + p_ref[1]) / cnt
    hid_ref[...] = pooled
    out_ref[...] = jnp.dot(pooled, wo_ref[...],
                           preferred_element_type=jnp.float32) + bo_ref[...]


_tc_head = pl.pallas_call(
    _tc_head_body,
    out_shape=(jax.ShapeDtypeStruct((NG, 1), jnp.float32),
               jax.ShapeDtypeStruct((NG, H), jnp.float32)),
)


def kernel(x, edge_index, batch_index, W0, b0, W1, b1, W2, b2, W3, b3,
           W_out, b_out):
    src = edge_index[0]
    dst = edge_index[1]
    pad_dst = N + (jnp.arange(E_PAD - E, dtype=jnp.int32) % (NROW - N))
    srcp = jnp.concatenate(
        [src, jnp.zeros((E_PAD - E,), jnp.int32)]).reshape(NW * ECK, ECH)
    dstp = jnp.concatenate([dst, pad_dst]).reshape(NW * ECK, ECH)
    pad_bid = NG + (jnp.arange(N_PAD - N, dtype=jnp.int32) % (GROW - NG - 8))
    bidw = jnp.concatenate(
        [batch_index, pad_bid]).reshape(NW, PCK, PCH)
    bidp = jnp.concatenate(
        [bidw, jnp.full((NW, 8 - PCK, PCH), NG, jnp.int32)],
        axis=1).reshape(NW * 8, PCH)
    z64 = jnp.zeros((ZR, H), jnp.float32)
    z16 = jnp.zeros((ZR, 16), jnp.float32)
    ones16 = jnp.ones((ECH, 16), jnp.float32)

    degp = _sc_degree(dstp, z16, ones16)
    g, dinv = _tc_g0(x, W0, degp)

    bs = (b0, b1, b2, b3)
    nxt = (W1, W2, W3)
    for l in range(4):
        S = _sc_propagate(g, srcp, dstp, z64)
        if l < 3:
            g = _tc_epi(S, g, dinv, bs[l].reshape(1, H), nxt[l])
        else:
            h3 = _tc_last(S, g, dinv, bs[3].reshape(1, H))

    h3p = jnp.pad(h3, ((0, N_PAD - N), (0, 0)))
    P, C = _sc_pool(h3p, bidp, z64, z16, ones16)
    out, hid = _tc_head(P, C, W_out, b_out.reshape(1, 1))
    return (out, hid)


# speedup vs baseline: 1.2866x; 1.2779x over previous
import functools

import jax
import jax.numpy as jnp
from jax import lax
from jax.experimental import pallas as pl
from jax.experimental.pallas import tpu as pltpu
from jax.experimental.pallas import tpu_sc as plsc

N = 10000
E = 320000
DIN = 128
H = 64
NG = 256

NC = 2
NS = 16
NW = NC * NS

ECH = 128
ECK = 80
EPW = ECH * ECK
E_PAD = EPW * NW

NROW = N + 112
ZR = NROW // NS

NBUF = 4
LOOK = 2

PCH = 64
PCK = 5
NPW = PCH * PCK
N_PAD = NPW * NW
GROW = 384
GZ = GROW // NS
GW = NG // NS

_mesh = plsc.VectorSubcoreMesh(core_axis_name="c", subcore_axis_name="s")
_sc_params = pltpu.CompilerParams(use_tc_tiling_on_sc=False)


@functools.partial(
    pl.kernel,
    out_type=jax.ShapeDtypeStruct((NC, NROW, 16), jnp.float32),
    mesh=_mesh,
    scratch_types=[
        pltpu.VMEM((ECK, ECH), jnp.int32),
        pltpu.VMEM((ECH, 16), jnp.float32),
        pltpu.VMEM_SHARED((NROW, 16), jnp.float32),
    ],
    compiler_params=_sc_params,
)
def _sc_degree(dstp_hbm, z16_hbm, ones_hbm, out_hbm, didx, ones, acc):
    cid = lax.axis_index("c")
    sid = lax.axis_index("s")
    wid = cid * NS + sid
    pltpu.sync_copy(z16_hbm.at[pl.ds(sid * ZR, ZR)], acc.at[pl.ds(sid * ZR, ZR)])
    pltpu.sync_copy(ones_hbm, ones)
    pltpu.sync_copy(dstp_hbm.at[pl.ds(wid * ECK, ECK)], didx)
    plsc.subcore_barrier()

    def body(j, carry):
        pltpu.sync_copy(ones, acc.at[didx.at[j]], add=True)
        return carry

    lax.fori_loop(0, ECK, body, 0)
    plsc.subcore_barrier()
    pltpu.sync_copy(acc.at[pl.ds(sid * ZR, ZR)],
                    out_hbm.at[cid, pl.ds(sid * ZR, ZR)])


@functools.partial(
    pl.kernel,
    out_type=jax.ShapeDtypeStruct((NC, NROW, H), jnp.float32),
    mesh=_mesh,
    scratch_types=[
        pltpu.VMEM((ECK, ECH), jnp.int32),
        pltpu.VMEM((ECK, ECH), jnp.int32),
        [pltpu.VMEM((ECH, H), jnp.float32)] * NBUF,
        [pltpu.SemaphoreType.DMA] * NBUF,
        [pltpu.SemaphoreType.DMA] * NBUF,
        pltpu.VMEM_SHARED((NROW, H), jnp.float32),
    ],
    compiler_params=_sc_params,
)
def _sc_propagate(g_hbm, srcp_hbm, dstp_hbm, z64_hbm, out_hbm,
                  sidx, didx, rows, semg, sems, acc):
    cid = lax.axis_index("c")
    sid = lax.axis_index("s")
    wid = cid * NS + sid
    pltpu.sync_copy(z64_hbm.at[pl.ds(sid * ZR, ZR)], acc.at[pl.ds(sid * ZR, ZR)])
    pltpu.sync_copy(srcp_hbm.at[pl.ds(wid * ECK, ECK)], sidx)
    pltpu.sync_copy(dstp_hbm.at[pl.ds(wid * ECK, ECK)], didx)
    plsc.subcore_barrier()

    for k in range(LOOK):
        pltpu.async_copy(g_hbm.at[sidx.at[k]], rows[k], semg[k])

    def body(i, carry):
        for k in range(NBUF):
            j = i * NBUF + k
            kn = (k + LOOK) % NBUF
            pltpu.make_async_copy(g_hbm.at[sidx.at[j]], rows[k], semg[k]).wait()

            @pl.when(j >= LOOK)
            def _():
                pltpu.make_async_copy(
                    rows[kn], acc.at[didx.at[j]], sems[kn]).wait()

            pltpu.async_copy(rows[k], acc.at[didx.at[j]], sems[k], add=True)

            @pl.when(j + LOOK < ECK)
            def _():
                pltpu.async_copy(g_hbm.at[sidx.at[j + LOOK]], rows[kn], semg[kn])
        return carry

    lax.fori_loop(0, ECK // NBUF, body, 0)
    for k in range(NBUF - LOOK, NBUF):
        pltpu.make_async_copy(rows[k], acc.at[didx.at[0]], sems[k]).wait()
    plsc.subcore_barrier()
    pltpu.sync_copy(acc.at[pl.ds(sid * ZR, ZR)],
                    out_hbm.at[cid, pl.ds(sid * ZR, ZR)])


@functools.partial(
    pl.kernel,
    out_type=(jax.ShapeDtypeStruct((NC, NG, H), jnp.float32),
              jax.ShapeDtypeStruct((NC, NG, 16), jnp.float32)),
    mesh=_mesh,
    scratch_types=[
        pltpu.VMEM((8, PCH), jnp.int32),
        pltpu.VMEM((PCH, H), jnp.float32),
        pltpu.VMEM((PCH, 16), jnp.float32),
        pltpu.VMEM_SHARED((GROW, H), jnp.float32),
        pltpu.VMEM_SHARED((GROW, 16), jnp.float32),
    ],
    compiler_params=_sc_params,
)
def _sc_pool(h_hbm, bidp_hbm, z64_hbm, z16_hbm, ones_hbm, outp_hbm, outc_hbm,
             bidx, rows, ones, accp, accc):
    cid = lax.axis_index("c")
    sid = lax.axis_index("s")
    wid = cid * NS + sid
    pltpu.sync_copy(z64_hbm.at[pl.ds(0, GZ)], accp.at[pl.ds(sid * GZ, GZ)])
    pltpu.sync_copy(z16_hbm.at[pl.ds(0, GZ)], accc.at[pl.ds(sid * GZ, GZ)])
    pltpu.sync_copy(ones_hbm.at[pl.ds(0, PCH)], ones)
    pltpu.sync_copy(bidp_hbm.at[pl.ds(wid * 8, 8)], bidx)
    plsc.subcore_barrier()

    def body(j, carry):
        pltpu.sync_copy(h_hbm.at[pl.ds(wid * NPW + j * PCH, PCH)], rows)
        pltpu.sync_copy(rows, accp.at[bidx.at[j]], add=True)
        pltpu.sync_copy(ones, accc.at[bidx.at[j]], add=True)
        return carry

    lax.fori_loop(0, PCK, body, 0)
    plsc.subcore_barrier()
    pltpu.sync_copy(accp.at[pl.ds(sid * GW, GW)],
                    outp_hbm.at[cid, pl.ds(sid * GW, GW)])
    pltpu.sync_copy(accc.at[pl.ds(sid * GW, GW)],
                    outc_hbm.at[cid, pl.ds(sid * GW, GW)])


def _tc_g0_body(x_ref, w_ref, deg_ref, g_ref, dinv_ref):
    deg = deg_ref[0, :N, 0:1] + deg_ref[1, :N, 0:1] + 1.0
    dinv = lax.rsqrt(deg)
    g_ref[...] = dinv * jnp.dot(x_ref[...], w_ref[...],
                                preferred_element_type=jnp.float32)
    dinv_ref[...] = dinv


_tc_g0 = pl.pallas_call(
    _tc_g0_body,
    out_shape=(jax.ShapeDtypeStruct((N, H), jnp.float32),
               jax.ShapeDtypeStruct((N, 1), jnp.float32)),
)


def _tc_epi_body(s_ref, g_ref, dinv_ref, b_ref, w_ref, out_ref):
    dinv = dinv_ref[...]
    h = jnp.maximum(
        dinv * (s_ref[0, :N] + s_ref[1, :N] + g_ref[...]) + b_ref[...], 0.0)
    out_ref[...] = dinv * jnp.dot(h, w_ref[...],
                                  preferred_element_type=jnp.float32)


_tc_epi = pl.pallas_call(
    _tc_epi_body,
    out_shape=jax.ShapeDtypeStruct((N, H), jnp.float32),
)


def _tc_last_body(s_ref, g_ref, dinv_ref, b_ref, out_ref):
    dinv = dinv_ref[...]
    out_ref[...] = jnp.maximum(
        dinv * (s_ref[0, :N] + s_ref[1, :N] + g_ref[...]) + b_ref[...], 0.0)


_tc_last = pl.pallas_call(
    _tc_last_body,
    out_shape=jax.ShapeDtypeStruct((N, H), jnp.float32),
)


def _tc_head_body(p_ref, c_ref, wo_ref, bo_ref, out_ref, hid_ref):
    cnt = jnp.maximum(c_ref[0, :, 0:1] + c_ref[1, :, 0:1], 1.0)
    pooled = (p_ref[0] + p_ref[1]) / cnt
    hid_ref[...] = pooled
    out_ref[...] = jnp.dot(pooled, wo_ref[...],
                           preferred_element_type=jnp.float32) + bo_ref[...]


_tc_head = pl.pallas_call(
    _tc_head_body,
    out_shape=(jax.ShapeDtypeStruct((NG, 1), jnp.float32),
               jax.ShapeDtypeStruct((NG, H), jnp.float32)),
)


def kernel(x, edge_index, batch_index, W0, b0, W1, b1, W2, b2, W3, b3,
           W_out, b_out):
    src = edge_index[0]
    dst = edge_index[1]
    pad_dst = N + (jnp.arange(E_PAD - E, dtype=jnp.int32) % (NROW - N))
    srcp = jnp.concatenate(
        [src, jnp.zeros((E_PAD - E,), jnp.int32)]).reshape(NW * ECK, ECH)
    dstp = jnp.concatenate([dst, pad_dst]).reshape(NW * ECK, ECH)
    pad_bid = NG + (jnp.arange(N_PAD - N, dtype=jnp.int32) % (GROW - NG - 8))
    bidw = jnp.concatenate(
        [batch_index, pad_bid]).reshape(NW, PCK, PCH)
    bidp = jnp.concatenate(
        [bidw, jnp.full((NW, 8 - PCK, PCH), NG, jnp.int32)],
        axis=1).reshape(NW * 8, PCH)
    z64 = jnp.zeros((NROW, H), jnp.float32)
    z16 = jnp.zeros((NROW, 16), jnp.float32)
    ones16 = jnp.ones((ECH, 16), jnp.float32)

    degp = _sc_degree(dstp, z16, ones16)
    g, dinv = _tc_g0(x, W0, degp)

    bs = (b0, b1, b2, b3)
    nxt = (W1, W2, W3)
    for l in range(4):
        S = _sc_propagate(g, srcp, dstp, z64)
        if l < 3:
            g = _tc_epi(S, g, dinv, bs[l].reshape(1, H), nxt[l])
        else:
            h3 = _tc_last(S, g, dinv, bs[3].reshape(1, H))

    h3p = jnp.pad(h3, ((0, N_PAD - N), (0, 0)))
    P, C = _sc_pool(h3p, bidp, z64, z16, ones16)
    out, hid = _tc_head(P, C, W_out, b_out.reshape(1, 1))
    return (out, hid)
